# Initial kernel scaffold; baseline (speedup 1.0000x reference)
import jax
import jax.numpy as jnp
from jax.experimental import pallas as pl


def kernel(dfb_w, dfb_u, c2_t0, c2_t1, c4_t0, c4_t1, c8_t0, c8_t1, c16_t0, c16_t1, c32_t0, c32_t1, c64_t0, c64_t1, d4_c1_w, d4_c1_u, d4_bn1_g, d4_bn1_b, d4_c2_w, d4_c2_u, d4_bn2_g, d4_bn2_b, d4_c3_w, d4_c3_u, d4_bn3_g, d4_bn3_b, d8_c1_w, d8_c1_u, d8_bn1_g, d8_bn1_b, d8_c2_w, d8_c2_u, d8_bn2_g, d8_bn2_b, d8_c3_w, d8_c3_u, d8_bn3_g, d8_bn3_b, d16_c1_w, d16_c1_u, d16_bn1_g, d16_bn1_b, d16_c2_w, d16_c2_u, d16_bn2_g, d16_bn2_b, d16_c3_w, d16_c3_u, d16_bn3_g, d16_bn3_b, d32_c1_w, d32_c1_u, d32_bn1_g, d32_bn1_b, d32_c2_w, d32_c2_u, d32_bn2_g, d32_bn2_b, d32_c3_w, d32_c3_u, d32_bn3_g, d32_bn3_b, d64_c1_w, d64_c1_u, d64_bn1_g, d64_bn1_b, d64_c2_w, d64_c2_u, d64_bn2_g, d64_bn2_b, d64_c3_w, d64_c3_u, d64_bn3_g, d64_bn3_b, rfb_c1_w, rfb_c1_u, rfb_bn1_g, rfb_bn1_b, rfb_c2_w, rfb_c2_u, se216_c1_w, se216_c1_u, se216_c2_w, se216_c2_u, se432_c1_w, se432_c1_u, se432_c2_w, se432_c2_u, se864_c1_w, se864_c1_u, se864_c2_w, se864_c2_u, dfs_w, dfs_u, c4s_t0, c4s_t1, c8s_t0, c8s_t1, c16s_t0, c16s_t1, c32s_t0, c32s_t1, d8s_c1_w, d8s_c1_u, d8s_bn1_g, d8s_bn1_b, d16s_c1_w, d16s_c1_u, d16s_bn1_g, d16s_bn1_b, d32s_c1_w, d32s_c1_u, d32s_bn1_g, d32s_bn1_b, rfs_w, rfs_u, decb_up0_c_w, decb_up0_c_u, decb_up0_bn_g, decb_up0_bn_b, decb_up1_c_w, decb_up1_c_u, decb_up1_bn_g, decb_up1_bn_b, decb_up2_c_w, decb_up2_c_u, decb_up2_bn_g, decb_up2_bn_b, decb_up3_c_w, decb_up3_c_u, decb_up3_bn_g, decb_up3_bn_b, decb_cout_w, decb_cout_u, decs_up0_c_w, decs_up0_c_u, decs_up0_bn_g, decs_up0_bn_b, decs_up1_c_w, decs_up1_c_u, decs_up1_bn_g, decs_up1_bn_b, decs_up2_c_w, decs_up2_c_u, decs_up2_bn_g, decs_up2_bn_b, decs_up3_c_w, decs_up3_c_u, decs_up3_bn_g, decs_up3_bn_b, decs_cout_w, decs_cout_u, decp_up0_c_w, decp_up0_c_u, decp_up0_bn_g, decp_up0_bn_b, decp_up1_c_w, decp_up1_c_u, decp_up1_bn_g, decp_up1_bn_b, decp_up2_c_w, decp_up2_c_u, decp_up2_bn_g, decp_up2_bn_b, decp_up3_c_w, decp_up3_c_u, decp_up3_bn_g, decp_up3_bn_b, decp_cout_w, decp_cout_u, imgs, y):
    raise NotImplementedError("write your pallas kernel here")


# speedup vs baseline: 2.8065x; 2.8065x over previous
import functools

import jax
import jax.numpy as jnp
from jax.experimental import pallas as pl
from jax.experimental.pallas import tpu as pltpu


def _act_fn(x, act):
    if act == "none":
        return x
    if act == "leaky":
        return jnp.where(x >= 0, x, 0.2 * x)
    if act == "swish":
        return x * jax.nn.sigmoid(x)
    if act == "sigmoid":
        return jax.nn.sigmoid(x)
    if act == "tanh":
        return jnp.tanh(x)
    raise ValueError(f"unknown act {act}")


def _tap_conv_kernel(x_ref, w_ref, o_ref, *, kh, kw, act):
    _, ho, wo, oc = o_ref.shape
    c = x_ref.shape[3]
    acc = None
    for di in range(kh):
        for dj in range(kw):
            patch = x_ref[0, di:di + ho, dj:dj + wo, :].reshape(ho * wo, c)
            t = jnp.dot(patch, w_ref[di * kw + dj],
                        preferred_element_type=jnp.float32)
            acc = t if acc is None else acc + t
    o_ref[0, :, :, :] = _act_fn(acc, act).reshape(ho, wo, oc)


_CONV_CACHE = {}


def _tap_conv(x, wt, kh, kw, act):
    g, hp, wp, c = x.shape
    o = wt.shape[2]
    ho, wo = hp - kh + 1, wp - kw + 1
    key = (g, hp, wp, c, o, kh, kw, act)
    fn = _CONV_CACHE.get(key)
    if fn is None:
        fn = pl.pallas_call(
            functools.partial(_tap_conv_kernel, kh=kh, kw=kw, act=act),
            out_shape=jax.ShapeDtypeStruct((g, ho, wo, o), jnp.float32),
            grid=(g,),
            in_specs=[
                pl.BlockSpec((1, hp, wp, c), lambda i: (i, 0, 0, 0)),
                pl.BlockSpec((kh * kw, c, o), lambda i: (0, 0, 0)),
            ],
            out_specs=pl.BlockSpec((1, ho, wo, o), lambda i: (i, 0, 0, 0)),
            compiler_params=pltpu.CompilerParams(
                dimension_semantics=("parallel",)),
        )
        _CONV_CACHE[key] = fn
    return fn(x, wt)


def conv_nhwc(x, w, pad, act="none", row_split=1):
    b, h, wd, c = x.shape
    o, _, kh, kw = w.shape
    if pad:
        x = jnp.pad(x, ((0, 0), (pad, pad), (pad, pad), (0, 0)))
    wt = jnp.transpose(w, (2, 3, 1, 0)).reshape(kh * kw, c, o)
    if row_split == 1 and h >= 128:
        row_split = 8 if h >= 256 else 2
    if row_split == 1:
        return _tap_conv(x, wt, kh, kw, act)
    ns = row_split
    sh = h // ns
    halo = kh - 1
    strips = jnp.stack([x[:, i * sh:i * sh + sh + halo] for i in range(ns)], 1)
    strips = strips.reshape(b * ns, sh + halo, wd + 2 * pad, c)
    y = _tap_conv(strips, wt, kh, kw, act)
    return y.reshape(b, ns * sh, wd + 2 * pad - kw + 1, o)


def conv_s2_nhwc(x, w, act="none"):
    b, h, wd, c = x.shape
    o = w.shape[0]
    xp = jnp.pad(x, ((0, 0), (1, 1), (1, 1), (0, 0)))
    h2, w2 = (h + 2) // 2, (wd + 2) // 2
    s = xp.reshape(b, h2, 2, w2, 2, c)
    s = s.transpose(0, 1, 3, 2, 4, 5).reshape(b, h2, w2, 4 * c)
    wt = jnp.transpose(w, (2, 3, 1, 0))
    wt = wt.reshape(2, 2, 2, 2, c, o)
    wt = wt.transpose(0, 2, 1, 3, 4, 5).reshape(4, 4 * c, o)
    return _tap_conv(s, wt, 2, 2, act)


def _mm_kernel(a_ref, b_ref, o_ref, *, act):
    o_ref[...] = _act_fn(
        jnp.dot(a_ref[...], b_ref[...], preferred_element_type=jnp.float32),
        act)


_MM_CACHE = {}


def matmul_act(a, b, act="none"):
    m0, k = a.shape
    n = b.shape[1]
    m = ((m0 + 7) // 8) * 8
    if m != m0:
        a = jnp.pad(a, ((0, m - m0), (0, 0)))
    key = (m, k, n, act)
    fn = _MM_CACHE.get(key)
    if fn is None:
        fn = pl.pallas_call(
            functools.partial(_mm_kernel, act=act),
            out_shape=jax.ShapeDtypeStruct((m, n), jnp.float32),
        )
        _MM_CACHE[key] = fn
    out = fn(a, b)
    return out[:m0] if m != m0 else out


def _leaky(x):
    return jnp.where(x >= 0, x, 0.2 * x)


def _batch_norm(x, g, b, eps=1e-5):
    mean = jnp.mean(x, axis=(0, 1, 2), keepdims=True)
    var = jnp.var(x, axis=(0, 1, 2), keepdims=True)
    return (x - mean) * jax.lax.rsqrt(var + eps) * g + b


def _spec_norm(w, u):
    o = w.shape[0]
    wm = w.reshape(o, -1)

    def l2n(v):
        return v / (jnp.linalg.norm(v) + 1e-12)

    v = l2n(wm.T @ u)
    u2 = l2n(wm @ v)
    sigma = u2 @ (wm @ v)
    return w / sigma


def _avg_pool_2x2(x):
    b, h, w, c = x.shape
    return x.reshape(b, h // 2, 2, w // 2, 2, c).mean(axis=(2, 4))


def _adaptive_pool(x, out):
    b, h, w, c = x.shape
    return x.reshape(b, out, h // out, out, w // out, c).mean(axis=(2, 4))


def _upsample2(x):
    return jnp.repeat(jnp.repeat(x, 2, axis=1), 2, axis=2)


def _cond_encode(t0, t1, y, cd):
    i0 = jnp.where(y[:, 0] == -1, 2, y[:, 0])
    i1 = jnp.where(y[:, 1] == -1, 3, y[:, 1])
    e = jnp.concatenate([jnp.take(t0, i0, axis=0),
                         jnp.take(t1, i1, axis=0)], axis=1)
    return e.reshape(y.shape[0], -1, cd, cd).transpose(0, 2, 3, 1)


def _sn_conv(w, u, x, pad, act="none", row_split=1):
    return conv_nhwc(x, _spec_norm(w, u), pad, act=act, row_split=row_split)


def _sn_conv_s2(w, u, x, act="none"):
    return conv_s2_nhwc(x, _spec_norm(w, u), act=act)


def _down_block_comp(x, c1_w, c1_u, bn1_g, bn1_b, c2_w, c2_u, bn2_g, bn2_b,
                     c3_w, c3_u, bn3_g, bn3_b):
    m = _sn_conv_s2(c1_w, c1_u, x)
    m = _leaky(_batch_norm(m, bn1_g, bn1_b))
    m = _sn_conv(c2_w, c2_u, m, 1)
    m = _leaky(_batch_norm(m, bn2_g, bn2_b))
    d = _avg_pool_2x2(x)
    d = _sn_conv(c3_w, c3_u, d, 0)
    d = _leaky(_batch_norm(d, bn3_g, bn3_b))
    return (m + d) * 0.5


def _down_block(x, c1_w, c1_u, bn1_g, bn1_b):
    y = _sn_conv_s2(c1_w, c1_u, x)
    return _leaky(_batch_norm(y, bn1_g, bn1_b))


def _head_conv_k4(x, w, u):
    b = x.shape[0]
    c = x.shape[3]
    wn = _spec_norm(w, u)
    taps = [x[:, di:di + 5, dj:dj + 5, :] for di in range(4) for dj in range(4)]
    p = jnp.concatenate(taps, axis=3).reshape(b * 25, 16 * c)
    wm = jnp.transpose(wn, (2, 3, 1, 0)).reshape(16 * c, 1)
    return matmul_act(p, wm).reshape(b, 5, 5, 1)


def _se_block(x_small, x_big, c1_w, c1_u, c2_w, c2_u):
    b = x_small.shape[0]
    c = x_small.shape[3]
    s = _adaptive_pool(x_small, 4).reshape(b, 16 * c)
    w1 = _spec_norm(c1_w, c1_u)
    m1 = jnp.transpose(w1, (2, 3, 1, 0)).reshape(16 * c, -1)
    s = matmul_act(s, m1, act="swish")
    w2 = _spec_norm(c2_w, c2_u)
    s = matmul_act(s, w2.reshape(w2.shape[0], -1).T, act="sigmoid")
    return x_big * s[:, None, None, :]


def _glu(x):
    c = x.shape[3] // 2
    return x[..., :c] * jax.nn.sigmoid(x[..., c:])


def _decoder(x, ups, cout_w, cout_u):
    y = x
    for (c_w, c_u, bn_g, bn_b) in ups:
        y = _upsample2(y)
        y = _sn_conv(c_w, c_u, y, 1)
        y = _batch_norm(y, bn_g, bn_b)
        y = _glu(y)
    y = _sn_conv(cout_w, cout_u, y, 1, act="tanh")
    return y.transpose(0, 3, 1, 2)


def kernel(dfb_w, dfb_u, c2_t0, c2_t1, c4_t0, c4_t1, c8_t0, c8_t1, c16_t0, c16_t1, c32_t0, c32_t1, c64_t0, c64_t1, d4_c1_w, d4_c1_u, d4_bn1_g, d4_bn1_b, d4_c2_w, d4_c2_u, d4_bn2_g, d4_bn2_b, d4_c3_w, d4_c3_u, d4_bn3_g, d4_bn3_b, d8_c1_w, d8_c1_u, d8_bn1_g, d8_bn1_b, d8_c2_w, d8_c2_u, d8_bn2_g, d8_bn2_b, d8_c3_w, d8_c3_u, d8_bn3_g, d8_bn3_b, d16_c1_w, d16_c1_u, d16_bn1_g, d16_bn1_b, d16_c2_w, d16_c2_u, d16_bn2_g, d16_bn2_b, d16_c3_w, d16_c3_u, d16_bn3_g, d16_bn3_b, d32_c1_w, d32_c1_u, d32_bn1_g, d32_bn1_b, d32_c2_w, d32_c2_u, d32_bn2_g, d32_bn2_b, d32_c3_w, d32_c3_u, d32_bn3_g, d32_bn3_b, d64_c1_w, d64_c1_u, d64_bn1_g, d64_bn1_b, d64_c2_w, d64_c2_u, d64_bn2_g, d64_bn2_b, d64_c3_w, d64_c3_u, d64_bn3_g, d64_bn3_b, rfb_c1_w, rfb_c1_u, rfb_bn1_g, rfb_bn1_b, rfb_c2_w, rfb_c2_u, se216_c1_w, se216_c1_u, se216_c2_w, se216_c2_u, se432_c1_w, se432_c1_u, se432_c2_w, se432_c2_u, se864_c1_w, se864_c1_u, se864_c2_w, se864_c2_u, dfs_w, dfs_u, c4s_t0, c4s_t1, c8s_t0, c8s_t1, c16s_t0, c16s_t1, c32s_t0, c32s_t1, d8s_c1_w, d8s_c1_u, d8s_bn1_g, d8s_bn1_b, d16s_c1_w, d16s_c1_u, d16s_bn1_g, d16s_bn1_b, d32s_c1_w, d32s_c1_u, d32s_bn1_g, d32s_bn1_b, rfs_w, rfs_u, decb_up0_c_w, decb_up0_c_u, decb_up0_bn_g, decb_up0_bn_b, decb_up1_c_w, decb_up1_c_u, decb_up1_bn_g, decb_up1_bn_b, decb_up2_c_w, decb_up2_c_u, decb_up2_bn_g, decb_up2_bn_b, decb_up3_c_w, decb_up3_c_u, decb_up3_bn_g, decb_up3_bn_b, decb_cout_w, decb_cout_u, decs_up0_c_w, decs_up0_c_u, decs_up0_bn_g, decs_up0_bn_b, decs_up1_c_w, decs_up1_c_u, decs_up1_bn_g, decs_up1_bn_b, decs_up2_c_w, decs_up2_c_u, decs_up2_bn_g, decs_up2_bn_b, decs_up3_c_w, decs_up3_c_u, decs_up3_bn_g, decs_up3_bn_b, decs_cout_w, decs_cout_u, decp_up0_c_w, decp_up0_c_u, decp_up0_bn_g, decp_up0_bn_b, decp_up1_c_w, decp_up1_c_u, decp_up1_bn_g, decp_up1_bn_b, decp_up2_c_w, decp_up2_c_u, decp_up2_bn_g, decp_up2_bn_b, decp_up3_c_w, decp_up3_c_u, decp_up3_bn_g, decp_up3_bn_b, decp_cout_w, decp_cout_u, imgs, y):
    imgs = imgs.transpose(0, 2, 3, 1)
    imgs_small = imgs[:, ::2, ::2, :]

    feat_2 = _sn_conv(dfb_w, dfb_u, imgs, 1, act="leaky")
    feat_2 = jnp.concatenate([feat_2, _cond_encode(c2_t0, c2_t1, y, 256)], -1)
    feat_4 = _down_block_comp(feat_2, d4_c1_w, d4_c1_u, d4_bn1_g, d4_bn1_b,
                              d4_c2_w, d4_c2_u, d4_bn2_g, d4_bn2_b,
                              d4_c3_w, d4_c3_u, d4_bn3_g, d4_bn3_b)
    feat_4 = jnp.concatenate([feat_4, _cond_encode(c4_t0, c4_t1, y, 128)], -1)
    feat_8 = _down_block_comp(feat_4, d8_c1_w, d8_c1_u, d8_bn1_g, d8_bn1_b,
                              d8_c2_w, d8_c2_u, d8_bn2_g, d8_bn2_b,
                              d8_c3_w, d8_c3_u, d8_bn3_g, d8_bn3_b)
    feat_8 = jnp.concatenate([feat_8, _cond_encode(c8_t0, c8_t1, y, 64)], -1)
    feat_16 = _down_block_comp(feat_8, d16_c1_w, d16_c1_u, d16_bn1_g, d16_bn1_b,
                               d16_c2_w, d16_c2_u, d16_bn2_g, d16_bn2_b,
                               d16_c3_w, d16_c3_u, d16_bn3_g, d16_bn3_b)
    feat_16 = jnp.concatenate([feat_16, _cond_encode(c16_t0, c16_t1, y, 32)], -1)
    feat_16 = _se_block(feat_2, feat_16, se216_c1_w, se216_c1_u,
                        se216_c2_w, se216_c2_u)
    feat_32 = _down_block_comp(feat_16, d32_c1_w, d32_c1_u, d32_bn1_g, d32_bn1_b,
                               d32_c2_w, d32_c2_u, d32_bn2_g, d32_bn2_b,
                               d32_c3_w, d32_c3_u, d32_bn3_g, d32_bn3_b)
    feat_32 = jnp.concatenate([feat_32, _cond_encode(c32_t0, c32_t1, y, 16)], -1)
    feat_32 = _se_block(feat_4, feat_32, se432_c1_w, se432_c1_u,
                        se432_c2_w, se432_c2_u)
    feat_last = _down_block_comp(feat_32, d64_c1_w, d64_c1_u, d64_bn1_g, d64_bn1_b,
                                 d64_c2_w, d64_c2_u, d64_bn2_g, d64_bn2_b,
                                 d64_c3_w, d64_c3_u, d64_bn3_g, d64_bn3_b)
    feat_last = jnp.concatenate([feat_last, _cond_encode(c64_t0, c64_t1, y, 8)], -1)
    feat_last = _se_block(feat_8, feat_last, se864_c1_w, se864_c1_u,
                          se864_c2_w, se864_c2_u)

    r = _sn_conv(rfb_c1_w, rfb_c1_u, feat_last, 0)
    r = _leaky(_batch_norm(r, rfb_bn1_g, rfb_bn1_b))
    rf_0 = _head_conv_k4(r, rfb_c2_w, rfb_c2_u).reshape(-1)

    feat_4s = _sn_conv_s2(dfs_w, dfs_u, imgs_small, act="leaky")
    feat_4s = jnp.concatenate([feat_4s, _cond_encode(c4s_t0, c4s_t1, y, 64)], -1)
    feat_8s = _down_block(feat_4s, d8s_c1_w, d8s_c1_u, d8s_bn1_g, d8s_bn1_b)
    feat_8s = jnp.concatenate([feat_8s, _cond_encode(c8s_t0, c8s_t1, y, 32)], -1)
    feat_16s = _down_block(feat_8s, d16s_c1_w, d16s_c1_u, d16s_bn1_g, d16s_bn1_b)
    feat_16s = jnp.concatenate([feat_16s, _cond_encode(c16s_t0, c16s_t1, y, 16)], -1)
    feat_small = _down_block(feat_16s, d32s_c1_w, d32s_c1_u, d32s_bn1_g, d32s_bn1_b)
    feat_small = jnp.concatenate([feat_small, _cond_encode(c32s_t0, c32s_t1, y, 8)], -1)
    rf_1 = _head_conv_k4(feat_small, rfs_w, rfs_u).reshape(-1)

    rf = jnp.concatenate([rf_0, rf_1])

    rec_big = _decoder(
        feat_last,
        [(decb_up0_c_w, decb_up0_c_u, decb_up0_bn_g, decb_up0_bn_b),
         (decb_up1_c_w, decb_up1_c_u, decb_up1_bn_g, decb_up1_bn_b),
         (decb_up2_c_w, decb_up2_c_u, decb_up2_bn_g, decb_up2_bn_b),
         (decb_up3_c_w, decb_up3_c_u, decb_up3_bn_g, decb_up3_bn_b)],
        decb_cout_w, decb_cout_u)
    rec_small = _decoder(
        feat_small,
        [(decs_up0_c_w, decs_up0_c_u, decs_up0_bn_g, decs_up0_bn_b),
         (decs_up1_c_w, decs_up1_c_u, decs_up1_bn_g, decs_up1_bn_b),
         (decs_up2_c_w, decs_up2_c_u, decs_up2_bn_g, decs_up2_bn_b),
         (decs_up3_c_w, decs_up3_c_u, decs_up3_bn_g, decs_up3_bn_b)],
        decs_cout_w, decs_cout_u)
    rec_part = _decoder(
        feat_32[:, :8, :8, :],
        [(decp_up0_c_w, decp_up0_c_u, decp_up0_bn_g, decp_up0_bn_b),
         (decp_up1_c_w, decp_up1_c_u, decp_up1_bn_g, decp_up1_bn_b),
         (decp_up2_c_w, decp_up2_c_u, decp_up2_bn_g, decp_up2_bn_b),
         (decp_up3_c_w, decp_up3_c_u, decp_up3_bn_g, decp_up3_bn_b)],
        decp_cout_w, decp_cout_u)
    return rf, [rec_big, rec_small, rec_part]


# speedup vs baseline: 3.0400x; 1.0832x over previous
import functools

import jax
import jax.numpy as jnp
from jax.experimental import pallas as pl
from jax.experimental.pallas import tpu as pltpu


def _act_fn(x, act):
    if act == "none":
        return x
    if act == "leaky":
        return jnp.where(x >= 0, x, 0.2 * x)
    if act == "swish":
        return x * jax.nn.sigmoid(x)
    if act == "sigmoid":
        return jax.nn.sigmoid(x)
    if act == "tanh":
        return jnp.tanh(x)
    raise ValueError(f"unknown act {act}")


def _tap_conv_kernel(x_ref, w_ref, o_ref, s_ref, *, kh, kw, pad, act):
    _, ho, wo, oc = o_ref.shape
    _, h, w, c = x_ref.shape
    if pad:
        s_ref[...] = jnp.zeros(s_ref.shape, jnp.float32)
        s_ref[pad:pad + h, pad:pad + w, :] = x_ref[0]
    else:
        s_ref[...] = x_ref[0]
    acc = None
    for di in range(kh):
        for dj in range(kw):
            patch = s_ref[di:di + ho, dj:dj + wo, :].reshape(ho * wo, c)
            t = jnp.dot(patch, w_ref[di * kw + dj],
                        preferred_element_type=jnp.float32)
            acc = t if acc is None else acc + t
    o_ref[0, :, :, :] = _act_fn(acc, act).reshape(ho, wo, oc)


_CONV_CACHE = {}


def _tap_conv(x, wt, kh, kw, pad, act):
    g, h, w, c = x.shape
    o = wt.shape[2]
    hp, wp = h + 2 * pad, w + 2 * pad
    ho, wo = hp - kh + 1, wp - kw + 1
    key = (g, h, w, c, o, kh, kw, pad, act)
    fn = _CONV_CACHE.get(key)
    if fn is None:
        fn = pl.pallas_call(
            functools.partial(_tap_conv_kernel, kh=kh, kw=kw, pad=pad, act=act),
            out_shape=jax.ShapeDtypeStruct((g, ho, wo, o), jnp.float32),
            grid=(g,),
            in_specs=[
                pl.BlockSpec((1, h, w, c), lambda i: (i, 0, 0, 0)),
                pl.BlockSpec((kh * kw, c, o), lambda i: (0, 0, 0)),
            ],
            out_specs=pl.BlockSpec((1, ho, wo, o), lambda i: (i, 0, 0, 0)),
            scratch_shapes=[pltpu.VMEM((hp, wp, c), jnp.float32)],
            compiler_params=pltpu.CompilerParams(
                dimension_semantics=("parallel",)),
        )
        _CONV_CACHE[key] = fn
    return fn(x, wt)


def _up_conv_kernel(x_ref, w_ref, o_ref, s_ref, *, act):
    _, r, wc, c = x_ref.shape
    oc = o_ref.shape[4]
    s_ref[...] = jnp.zeros(s_ref.shape, jnp.float32)
    s_ref[1:r + 1, 1:wc + 1, :] = x_ref[0]
    for pi in range(2):
        acc = None
        for a in range(2):
            for dj in range(3):
                patch = s_ref[pi + a:pi + a + r, dj:dj + wc, :].reshape(r * wc, c)
                t = jnp.dot(patch, w_ref[pi, a * 3 + dj],
                            preferred_element_type=jnp.float32)
                acc = t if acc is None else acc + t
        o_ref[0, :, pi, :, :] = _act_fn(acc, act).reshape(r, wc, oc)


_UPCONV_CACHE = {}

def up_conv_nhwc(x, w, act="none"):
    b, r, _, c = x.shape
    o = w.shape[0]
    xc = jnp.repeat(x, 2, axis=2)
    wt = jnp.transpose(w, (2, 3, 1, 0))
    mix = jnp.array([[[1., 0., 0.], [0., 1., 1.]],
                     [[1., 1., 0.], [0., 0., 1.]]], jnp.float32)
    wr = jnp.einsum("pad,djco->pajco", mix, wt)
    wr = wr.reshape(2, 6, c, o)
    key = (b, r, c, o, act)
    fn = _UPCONV_CACHE.get(key)
    if fn is None:
        fn = pl.pallas_call(
            functools.partial(_up_conv_kernel, act=act),
            out_shape=jax.ShapeDtypeStruct((b, r, 2, 2 * r, o), jnp.float32),
            grid=(b,),
            in_specs=[
                pl.BlockSpec((1, r, 2 * r, c), lambda i: (i, 0, 0, 0)),
                pl.BlockSpec((2, 6, c, o), lambda i: (0, 0, 0, 0)),
            ],
            out_specs=pl.BlockSpec((1, r, 2, 2 * r, o),
                                   lambda i: (i, 0, 0, 0, 0)),
            scratch_shapes=[pltpu.VMEM((r + 2, 2 * r + 2, c), jnp.float32)],
            compiler_params=pltpu.CompilerParams(
                dimension_semantics=("parallel",)),
        )
        _UPCONV_CACHE[key] = fn
    y = fn(xc, wr)
    return y.reshape(b, 2 * r, 2 * r, o)


def conv_nhwc(x, w, pad, act="none", row_split=1):
    b, h, wd, c = x.shape
    o, _, kh, kw = w.shape
    wt = jnp.transpose(w, (2, 3, 1, 0)).reshape(kh * kw, c, o)
    if row_split == 1 and h >= 128:
        row_split = 8 if h >= 256 else 2
    if row_split == 1:
        return _tap_conv(x, wt, kh, kw, pad, act)
    xp = jnp.pad(x, ((0, 0), (pad, pad), (pad, pad), (0, 0)))
    ns = row_split
    sh = h // ns
    halo = kh - 1
    strips = jnp.stack([xp[:, i * sh:i * sh + sh + halo] for i in range(ns)], 1)
    strips = strips.reshape(b * ns, sh + halo, wd + 2 * pad, c)
    y = _tap_conv(strips, wt, kh, kw, 0, act)
    return y.reshape(b, ns * sh, wd + 2 * pad - kw + 1, o)


def conv_s2_nhwc(x, w, act="none"):
    b, h, wd, c = x.shape
    o = w.shape[0]
    xp = jnp.pad(x, ((0, 0), (1, 1), (1, 1), (0, 0)))
    h2, w2 = (h + 2) // 2, (wd + 2) // 2
    s = xp.reshape(b, h2, 2, w2, 2, c)
    s = s.transpose(0, 1, 3, 2, 4, 5).reshape(b, h2, w2, 4 * c)
    wt = jnp.transpose(w, (2, 3, 1, 0))
    wt = wt.reshape(2, 2, 2, 2, c, o)
    wt = wt.transpose(0, 2, 1, 3, 4, 5).reshape(4, 4 * c, o)
    return _tap_conv(s, wt, 2, 2, 0, act)


def _mm_kernel(a_ref, b_ref, o_ref, *, act):
    o_ref[...] = _act_fn(
        jnp.dot(a_ref[...], b_ref[...], preferred_element_type=jnp.float32),
        act)


_MM_CACHE = {}


def matmul_act(a, b, act="none"):
    m0, k = a.shape
    n = b.shape[1]
    m = ((m0 + 7) // 8) * 8
    if m != m0:
        a = jnp.pad(a, ((0, m - m0), (0, 0)))
    key = (m, k, n, act)
    fn = _MM_CACHE.get(key)
    if fn is None:
        fn = pl.pallas_call(
            functools.partial(_mm_kernel, act=act),
            out_shape=jax.ShapeDtypeStruct((m, n), jnp.float32),
        )
        _MM_CACHE[key] = fn
    out = fn(a, b)
    return out[:m0] if m != m0 else out


def _leaky(x):
    return jnp.where(x >= 0, x, 0.2 * x)


def _batch_norm(x, g, b, eps=1e-5):
    mean = jnp.mean(x, axis=(0, 1, 2), keepdims=True)
    var = jnp.var(x, axis=(0, 1, 2), keepdims=True)
    return (x - mean) * jax.lax.rsqrt(var + eps) * g + b


def _spec_norm(w, u):
    o = w.shape[0]
    wm = w.reshape(o, -1)

    def l2n(v):
        return v / (jnp.linalg.norm(v) + 1e-12)

    v = l2n(wm.T @ u)
    u2 = l2n(wm @ v)
    sigma = u2 @ (wm @ v)
    return w / sigma


def _avg_pool_2x2(x):
    b, h, w, c = x.shape
    return x.reshape(b, h // 2, 2, w // 2, 2, c).mean(axis=(2, 4))


def _adaptive_pool(x, out):
    b, h, w, c = x.shape
    return x.reshape(b, out, h // out, out, w // out, c).mean(axis=(2, 4))


def _upsample2(x):
    return jnp.repeat(jnp.repeat(x, 2, axis=1), 2, axis=2)


def _cond_encode(t0, t1, y, cd):
    i0 = jnp.where(y[:, 0] == -1, 2, y[:, 0])
    i1 = jnp.where(y[:, 1] == -1, 3, y[:, 1])
    e = jnp.concatenate([jnp.take(t0, i0, axis=0),
                         jnp.take(t1, i1, axis=0)], axis=1)
    return e.reshape(y.shape[0], -1, cd, cd).transpose(0, 2, 3, 1)


def _sn_conv(w, u, x, pad, act="none", row_split=1):
    return conv_nhwc(x, _spec_norm(w, u), pad, act=act, row_split=row_split)


def _sn_conv_s2(w, u, x, act="none"):
    return conv_s2_nhwc(x, _spec_norm(w, u), act=act)


def _down_block_comp(x, c1_w, c1_u, bn1_g, bn1_b, c2_w, c2_u, bn2_g, bn2_b,
                     c3_w, c3_u, bn3_g, bn3_b):
    m = _sn_conv_s2(c1_w, c1_u, x)
    m = _leaky(_batch_norm(m, bn1_g, bn1_b))
    m = _sn_conv(c2_w, c2_u, m, 1)
    m = _leaky(_batch_norm(m, bn2_g, bn2_b))
    d = _avg_pool_2x2(x)
    d = _sn_conv(c3_w, c3_u, d, 0)
    d = _leaky(_batch_norm(d, bn3_g, bn3_b))
    return (m + d) * 0.5


def _down_block(x, c1_w, c1_u, bn1_g, bn1_b):
    y = _sn_conv_s2(c1_w, c1_u, x)
    return _leaky(_batch_norm(y, bn1_g, bn1_b))


def _head_conv_k4(x, w, u):
    b = x.shape[0]
    c = x.shape[3]
    wn = _spec_norm(w, u)
    taps = [x[:, di:di + 5, dj:dj + 5, :] for di in range(4) for dj in range(4)]
    p = jnp.concatenate(taps, axis=3).reshape(b * 25, 16 * c)
    wm = jnp.transpose(wn, (2, 3, 1, 0)).reshape(16 * c, 1)
    return matmul_act(p, wm).reshape(b, 5, 5, 1)


def _se_block(x_small, x_big, c1_w, c1_u, c2_w, c2_u):
    b = x_small.shape[0]
    c = x_small.shape[3]
    s = _adaptive_pool(x_small, 4).reshape(b, 16 * c)
    w1 = _spec_norm(c1_w, c1_u)
    m1 = jnp.transpose(w1, (2, 3, 1, 0)).reshape(16 * c, -1)
    s = matmul_act(s, m1, act="swish")
    w2 = _spec_norm(c2_w, c2_u)
    s = matmul_act(s, w2.reshape(w2.shape[0], -1).T, act="sigmoid")
    return x_big * s[:, None, None, :]


def _glu(x):
    c = x.shape[3] // 2
    return x[..., :c] * jax.nn.sigmoid(x[..., c:])


def _decoder(x, ups, cout_w, cout_u):
    y = x
    for (c_w, c_u, bn_g, bn_b) in ups:
        y = up_conv_nhwc(y, _spec_norm(c_w, c_u))
        y = _batch_norm(y, bn_g, bn_b)
        y = _glu(y)
    y = _sn_conv(cout_w, cout_u, y, 1, act="tanh")
    return y.transpose(0, 3, 1, 2)


def kernel(dfb_w, dfb_u, c2_t0, c2_t1, c4_t0, c4_t1, c8_t0, c8_t1, c16_t0, c16_t1, c32_t0, c32_t1, c64_t0, c64_t1, d4_c1_w, d4_c1_u, d4_bn1_g, d4_bn1_b, d4_c2_w, d4_c2_u, d4_bn2_g, d4_bn2_b, d4_c3_w, d4_c3_u, d4_bn3_g, d4_bn3_b, d8_c1_w, d8_c1_u, d8_bn1_g, d8_bn1_b, d8_c2_w, d8_c2_u, d8_bn2_g, d8_bn2_b, d8_c3_w, d8_c3_u, d8_bn3_g, d8_bn3_b, d16_c1_w, d16_c1_u, d16_bn1_g, d16_bn1_b, d16_c2_w, d16_c2_u, d16_bn2_g, d16_bn2_b, d16_c3_w, d16_c3_u, d16_bn3_g, d16_bn3_b, d32_c1_w, d32_c1_u, d32_bn1_g, d32_bn1_b, d32_c2_w, d32_c2_u, d32_bn2_g, d32_bn2_b, d32_c3_w, d32_c3_u, d32_bn3_g, d32_bn3_b, d64_c1_w, d64_c1_u, d64_bn1_g, d64_bn1_b, d64_c2_w, d64_c2_u, d64_bn2_g, d64_bn2_b, d64_c3_w, d64_c3_u, d64_bn3_g, d64_bn3_b, rfb_c1_w, rfb_c1_u, rfb_bn1_g, rfb_bn1_b, rfb_c2_w, rfb_c2_u, se216_c1_w, se216_c1_u, se216_c2_w, se216_c2_u, se432_c1_w, se432_c1_u, se432_c2_w, se432_c2_u, se864_c1_w, se864_c1_u, se864_c2_w, se864_c2_u, dfs_w, dfs_u, c4s_t0, c4s_t1, c8s_t0, c8s_t1, c16s_t0, c16s_t1, c32s_t0, c32s_t1, d8s_c1_w, d8s_c1_u, d8s_bn1_g, d8s_bn1_b, d16s_c1_w, d16s_c1_u, d16s_bn1_g, d16s_bn1_b, d32s_c1_w, d32s_c1_u, d32s_bn1_g, d32s_bn1_b, rfs_w, rfs_u, decb_up0_c_w, decb_up0_c_u, decb_up0_bn_g, decb_up0_bn_b, decb_up1_c_w, decb_up1_c_u, decb_up1_bn_g, decb_up1_bn_b, decb_up2_c_w, decb_up2_c_u, decb_up2_bn_g, decb_up2_bn_b, decb_up3_c_w, decb_up3_c_u, decb_up3_bn_g, decb_up3_bn_b, decb_cout_w, decb_cout_u, decs_up0_c_w, decs_up0_c_u, decs_up0_bn_g, decs_up0_bn_b, decs_up1_c_w, decs_up1_c_u, decs_up1_bn_g, decs_up1_bn_b, decs_up2_c_w, decs_up2_c_u, decs_up2_bn_g, decs_up2_bn_b, decs_up3_c_w, decs_up3_c_u, decs_up3_bn_g, decs_up3_bn_b, decs_cout_w, decs_cout_u, decp_up0_c_w, decp_up0_c_u, decp_up0_bn_g, decp_up0_bn_b, decp_up1_c_w, decp_up1_c_u, decp_up1_bn_g, decp_up1_bn_b, decp_up2_c_w, decp_up2_c_u, decp_up2_bn_g, decp_up2_bn_b, decp_up3_c_w, decp_up3_c_u, decp_up3_bn_g, decp_up3_bn_b, decp_cout_w, decp_cout_u, imgs, y):
    imgs = imgs.transpose(0, 2, 3, 1)
    imgs_small = imgs[:, ::2, ::2, :]

    feat_2 = _sn_conv(dfb_w, dfb_u, imgs, 1, act="leaky")
    feat_2 = jnp.concatenate([feat_2, _cond_encode(c2_t0, c2_t1, y, 256)], -1)
    feat_4 = _down_block_comp(feat_2, d4_c1_w, d4_c1_u, d4_bn1_g, d4_bn1_b,
                              d4_c2_w, d4_c2_u, d4_bn2_g, d4_bn2_b,
                              d4_c3_w, d4_c3_u, d4_bn3_g, d4_bn3_b)
    feat_4 = jnp.concatenate([feat_4, _cond_encode(c4_t0, c4_t1, y, 128)], -1)
    feat_8 = _down_block_comp(feat_4, d8_c1_w, d8_c1_u, d8_bn1_g, d8_bn1_b,
                              d8_c2_w, d8_c2_u, d8_bn2_g, d8_bn2_b,
                              d8_c3_w, d8_c3_u, d8_bn3_g, d8_bn3_b)
    feat_8 = jnp.concatenate([feat_8, _cond_encode(c8_t0, c8_t1, y, 64)], -1)
    feat_16 = _down_block_comp(feat_8, d16_c1_w, d16_c1_u, d16_bn1_g, d16_bn1_b,
                               d16_c2_w, d16_c2_u, d16_bn2_g, d16_bn2_b,
                               d16_c3_w, d16_c3_u, d16_bn3_g, d16_bn3_b)
    feat_16 = jnp.concatenate([feat_16, _cond_encode(c16_t0, c16_t1, y, 32)], -1)
    feat_16 = _se_block(feat_2, feat_16, se216_c1_w, se216_c1_u,
                        se216_c2_w, se216_c2_u)
    feat_32 = _down_block_comp(feat_16, d32_c1_w, d32_c1_u, d32_bn1_g, d32_bn1_b,
                               d32_c2_w, d32_c2_u, d32_bn2_g, d32_bn2_b,
                               d32_c3_w, d32_c3_u, d32_bn3_g, d32_bn3_b)
    feat_32 = jnp.concatenate([feat_32, _cond_encode(c32_t0, c32_t1, y, 16)], -1)
    feat_32 = _se_block(feat_4, feat_32, se432_c1_w, se432_c1_u,
                        se432_c2_w, se432_c2_u)
    feat_last = _down_block_comp(feat_32, d64_c1_w, d64_c1_u, d64_bn1_g, d64_bn1_b,
                                 d64_c2_w, d64_c2_u, d64_bn2_g, d64_bn2_b,
                                 d64_c3_w, d64_c3_u, d64_bn3_g, d64_bn3_b)
    feat_last = jnp.concatenate([feat_last, _cond_encode(c64_t0, c64_t1, y, 8)], -1)
    feat_last = _se_block(feat_8, feat_last, se864_c1_w, se864_c1_u,
                          se864_c2_w, se864_c2_u)

    r = _sn_conv(rfb_c1_w, rfb_c1_u, feat_last, 0)
    r = _leaky(_batch_norm(r, rfb_bn1_g, rfb_bn1_b))
    rf_0 = _head_conv_k4(r, rfb_c2_w, rfb_c2_u).reshape(-1)

    feat_4s = _sn_conv_s2(dfs_w, dfs_u, imgs_small, act="leaky")
    feat_4s = jnp.concatenate([feat_4s, _cond_encode(c4s_t0, c4s_t1, y, 64)], -1)
    feat_8s = _down_block(feat_4s, d8s_c1_w, d8s_c1_u, d8s_bn1_g, d8s_bn1_b)
    feat_8s = jnp.concatenate([feat_8s, _cond_encode(c8s_t0, c8s_t1, y, 32)], -1)
    feat_16s = _down_block(feat_8s, d16s_c1_w, d16s_c1_u, d16s_bn1_g, d16s_bn1_b)
    feat_16s = jnp.concatenate([feat_16s, _cond_encode(c16s_t0, c16s_t1, y, 16)], -1)
    feat_small = _down_block(feat_16s, d32s_c1_w, d32s_c1_u, d32s_bn1_g, d32s_bn1_b)
    feat_small = jnp.concatenate([feat_small, _cond_encode(c32s_t0, c32s_t1, y, 8)], -1)
    rf_1 = _head_conv_k4(feat_small, rfs_w, rfs_u).reshape(-1)

    rf = jnp.concatenate([rf_0, rf_1])

    rec_big = _decoder(
        feat_last,
        [(decb_up0_c_w, decb_up0_c_u, decb_up0_bn_g, decb_up0_bn_b),
         (decb_up1_c_w, decb_up1_c_u, decb_up1_bn_g, decb_up1_bn_b),
         (decb_up2_c_w, decb_up2_c_u, decb_up2_bn_g, decb_up2_bn_b),
         (decb_up3_c_w, decb_up3_c_u, decb_up3_bn_g, decb_up3_bn_b)],
        decb_cout_w, decb_cout_u)
    rec_small = _decoder(
        feat_small,
        [(decs_up0_c_w, decs_up0_c_u, decs_up0_bn_g, decs_up0_bn_b),
         (decs_up1_c_w, decs_up1_c_u, decs_up1_bn_g, decs_up1_bn_b),
         (decs_up2_c_w, decs_up2_c_u, decs_up2_bn_g, decs_up2_bn_b),
         (decs_up3_c_w, decs_up3_c_u, decs_up3_bn_g, decs_up3_bn_b)],
        decs_cout_w, decs_cout_u)
    rec_part = _decoder(
        feat_32[:, :8, :8, :],
        [(decp_up0_c_w, decp_up0_c_u, decp_up0_bn_g, decp_up0_bn_b),
         (decp_up1_c_w, decp_up1_c_u, decp_up1_bn_g, decp_up1_bn_b),
         (decp_up2_c_w, decp_up2_c_u, decp_up2_bn_g, decp_up2_bn_b),
         (decp_up3_c_w, decp_up3_c_u, decp_up3_bn_g, decp_up3_bn_b)],
        decp_cout_w, decp_cout_u)
    return rf, [rec_big, rec_small, rec_part]
